# baseline (speedup 1.0000x reference)
import functools

import jax
import jax.numpy as jnp
from jax import lax
from jax.experimental import pallas as pl
from jax.experimental.pallas import tpu as pltpu
from jax.experimental.pallas import tpu_sc as plsc

N = 10000
E = 320000
D = 128
NDIV = 9
NHEADS = 2
K = NHEADS * NDIV
NB = 400
NBLK = N // NB
NPAD = 10240
RP = NPAD // 16
EP = E // 16
CH = 100
NCH = EP // CH
IB = 40
NIB = NCH // IB
ZR = 16


def _msg_body(f_ref, n_ref, w_ref, col_ref, rel_ref, o_ref, gi_ref):
    feat = f_ref[...] * n_ref[...]
    for h in range(NHEADS):
        o_ref[h] = jnp.dot(feat, w_ref[h], preferred_element_type=jnp.float32)
    g = col_ref[...] * NDIV + rel_ref[...]
    gi_ref[0] = g
    gi_ref[1] = g + NDIV * N


def _sc_body(mtab, gidx, rowi, out,
             g0_v, g1_v, s0_v, s1_v, rows0_v, rows1_v, acc,
             sem0, sem1, semi0, semi1):
    h = lax.axis_index("c")
    s = lax.axis_index("s")
    gbase = (h * 16 + s) * NCH
    sbase = s * NCH
    gbufs = (g0_v, g1_v)
    sbufs = (s0_v, s1_v)
    isems = (semi0, semi1)

    def idx_load(b, par):
        c0 = pltpu.async_copy(gidx.at[pl.ds(gbase + b * IB, IB)], gbufs[par],
                              isems[par])
        c1 = pltpu.async_copy(rowi.at[pl.ds(sbase + b * IB, IB)], sbufs[par],
                              isems[par])
        return c0, c1

    ld = idx_load(0, 0)

    def zrow(i, _):
        for j in range(D // 16):
            rows0_v[i, pl.ds(j * 16, 16)] = jnp.zeros((16,), jnp.float32)
        return 0
    lax.fori_loop(0, CH, zrow, 0)

    ZC = 96
    def zcopy(k0, _):
        pltpu.sync_copy(rows0_v.at[pl.ds(0, ZC)],
                        acc.at[pl.ds(s * RP + k0 * ZC, ZC)])
        return 0
    lax.fori_loop(0, RP // ZC, zcopy, 0)
    pltpu.sync_copy(rows0_v.at[pl.ds(0, RP % ZC)],
                    acc.at[pl.ds(s * RP + (RP // ZC) * ZC, RP % ZC)])
    plsc.subcore_barrier()

    for b in range(NIB):
        par = b % 2
        gv, sv = gbufs[par], sbufs[par]
        ld[0].wait()
        ld[1].wait()
        if b + 1 < NIB:
            ld = idx_load(b + 1, 1 - par)
        pltpu.async_copy(mtab.at[gv.at[0]], rows0_v, sem0)

        def pair(p, _):
            ci = p * 2
            pltpu.make_async_copy(mtab.at[gv.at[ci]], rows0_v, sem0).wait()
            pltpu.async_copy(mtab.at[gv.at[ci + 1]], rows1_v, sem1)
            pltpu.sync_copy(rows0_v, acc.at[sv.at[ci]], add=True)
            pltpu.make_async_copy(mtab.at[gv.at[ci + 1]], rows1_v, sem1).wait()

            @pl.when(ci + 2 < IB)
            def _():
                pltpu.async_copy(mtab.at[gv.at[ci + 2]], rows0_v, sem0)
            pltpu.sync_copy(rows1_v, acc.at[sv.at[ci + 1]], add=True)
            return 0
        lax.fori_loop(0, IB // 2, pair, 0)
    plsc.subcore_barrier()

    pltpu.sync_copy(acc.at[pl.ds(s * RP, RP)],
                    out.at[pl.ds(h * NPAD + s * RP, RP)])


def _final_body(a_ref, n_ref, o_ref):
    a = a_ref[...]
    nrm = n_ref[...] * (1.0 / NDIV)
    o_ref[...] = 0.5 * (jnp.maximum(a[0] * nrm, 0.0) + jnp.maximum(a[1] * nrm, 0.0))


def kernel(features, norm, W, edge_index, edge_relation):
    row = edge_index[0].astype(jnp.int32)
    col = edge_index[1].astype(jnp.int32)
    rel = edge_relation.astype(jnp.int32)

    wcat = W.transpose(0, 2, 1, 3).reshape(NHEADS, D, NDIV * D)
    mtab, gidx = pl.pallas_call(
        _msg_body,
        grid=(NBLK,),
        in_specs=[
            pl.BlockSpec((NB, D), lambda i: (i, 0)),
            pl.BlockSpec((NB, 1), lambda i: (i, 0)),
            pl.BlockSpec((NHEADS, D, NDIV * D), lambda i: (0, 0, 0)),
            pl.BlockSpec((E // D, D), lambda i: (0, 0)),
            pl.BlockSpec((E // D, D), lambda i: (0, 0)),
        ],
        out_specs=[
            pl.BlockSpec((NHEADS, NB, NDIV * D), lambda i: (0, i, 0)),
            pl.BlockSpec((2, E // D, D), lambda i: (0, 0, 0)),
        ],
        out_shape=[
            jax.ShapeDtypeStruct((NHEADS, N, NDIV * D), jnp.float32),
            jax.ShapeDtypeStruct((2, E // D, D), jnp.int32),
        ],
    )(features, norm, wcat,
      col.reshape(E // D, D), rel.reshape(E // D, D))

    mesh = plsc.VectorSubcoreMesh(core_axis_name="c", subcore_axis_name="s")
    acc = pl.kernel(
        _sc_body,
        out_type=jax.ShapeDtypeStruct((NHEADS * NPAD, D), jnp.float32),
        mesh=mesh,
        scratch_types=[
            pltpu.VMEM((IB, CH), jnp.int32),
            pltpu.VMEM((IB, CH), jnp.int32),
            pltpu.VMEM((IB, CH), jnp.int32),
            pltpu.VMEM((IB, CH), jnp.int32),
            pltpu.VMEM((CH, D), jnp.float32),
            pltpu.VMEM((CH, D), jnp.float32),
            pltpu.VMEM_SHARED((NPAD, D), jnp.float32),
            pltpu.SemaphoreType.DMA,
            pltpu.SemaphoreType.DMA,
            pltpu.SemaphoreType.DMA,
            pltpu.SemaphoreType.DMA,
        ],
    )(mtab.reshape(K * N, D), gidx.reshape(NHEADS * E // CH, CH),
      row.reshape(E // CH, CH))

    out = pl.pallas_call(
        _final_body,
        grid=(NBLK,),
        in_specs=[
            pl.BlockSpec((NHEADS, NB, D), lambda i: (0, i, 0)),
            pl.BlockSpec((NB, 1), lambda i: (i, 0)),
        ],
        out_specs=pl.BlockSpec((NB, D), lambda i: (i, 0)),
        out_shape=jax.ShapeDtypeStruct((N, D), jnp.float32),
    )(acc.reshape(NHEADS, NPAD, D), norm)
    return out


# speedup vs baseline: 1.2351x; 1.2351x over previous
import jax
import jax.numpy as jnp
from jax import lax
from jax.experimental import pallas as pl
from jax.experimental.pallas import tpu as pltpu
from jax.experimental.pallas import tpu_sc as plsc

N = 10000
E = 320000
D = 128
NDIV = 9
NHEADS = 2
K = NHEADS * NDIV
NB = 400
NBLK = N // NB
NPAD = 10240
RP = NPAD // 16
EP = E // 16
CH = 100
NCH = EP // CH
IB = 40
NIB = NCH // IB


def _msg_body(f_ref, n_ref, w_ref, col_ref, rel_ref, o_ref, gi_ref):
    feat = f_ref[...] * n_ref[...]
    for j in range(K):
        o_ref[j] = jnp.dot(feat, w_ref[j], preferred_element_type=jnp.float32)
    g = rel_ref[...] * N + col_ref[...]
    gi_ref[0] = g
    gi_ref[1] = g + NDIV * N


def _sc_body(mtab, gidx, rowi, out,
             g0_v, g1_v, s0_v, s1_v, rows0_v, rows1_v, acc,
             sem0, sem1, semi0, semi1):
    h = lax.axis_index("c")
    s = lax.axis_index("s")
    gbase = (h * 16 + s) * NCH
    sbase = s * NCH
    gbufs = (g0_v, g1_v)
    sbufs = (s0_v, s1_v)
    isems = (semi0, semi1)

    def idx_load(b, par):
        c0 = pltpu.async_copy(gidx.at[pl.ds(gbase + b * IB, IB)], gbufs[par],
                              isems[par])
        c1 = pltpu.async_copy(rowi.at[pl.ds(sbase + b * IB, IB)], sbufs[par],
                              isems[par])
        return c0, c1

    ld = idx_load(0, 0)

    def zrow(i, _):
        for j in range(D // 16):
            rows0_v[i, pl.ds(j * 16, 16)] = jnp.zeros((16,), jnp.float32)
        return 0
    lax.fori_loop(0, CH, zrow, 0)

    ZC = 96
    def zcopy(k0, _):
        pltpu.sync_copy(rows0_v.at[pl.ds(0, ZC)],
                        acc.at[pl.ds(s * RP + k0 * ZC, ZC)])
        return 0
    lax.fori_loop(0, RP // ZC, zcopy, 0)
    pltpu.sync_copy(rows0_v.at[pl.ds(0, RP % ZC)],
                    acc.at[pl.ds(s * RP + (RP // ZC) * ZC, RP % ZC)])
    plsc.subcore_barrier()

    for b in range(NIB):
        par = b % 2
        gv, sv = gbufs[par], sbufs[par]
        ld[0].wait()
        ld[1].wait()
        if b + 1 < NIB:
            ld = idx_load(b + 1, 1 - par)
        pltpu.async_copy(mtab.at[gv.at[0]], rows0_v, sem0)

        def pair(p, _):
            ci = p * 2
            pltpu.make_async_copy(mtab.at[gv.at[ci]], rows0_v, sem0).wait()
            pltpu.async_copy(mtab.at[gv.at[ci + 1]], rows1_v, sem1)
            pltpu.sync_copy(rows0_v, acc.at[sv.at[ci]], add=True)
            pltpu.make_async_copy(mtab.at[gv.at[ci + 1]], rows1_v, sem1).wait()

            @pl.when(ci + 2 < IB)
            def _():
                pltpu.async_copy(mtab.at[gv.at[ci + 2]], rows0_v, sem0)
            pltpu.sync_copy(rows1_v, acc.at[sv.at[ci + 1]], add=True)
            return 0
        lax.fori_loop(0, IB // 2, pair, 0)
    plsc.subcore_barrier()

    pltpu.sync_copy(acc.at[pl.ds(s * RP, RP)],
                    out.at[pl.ds(h * NPAD + s * RP, RP)])


def _final_body(a_ref, n_ref, o_ref):
    a = a_ref[...]
    nrm = n_ref[...] * (1.0 / NDIV)
    o_ref[...] = 0.5 * (jnp.maximum(a[0] * nrm, 0.0) + jnp.maximum(a[1] * nrm, 0.0))


def kernel(features, norm, W, edge_index, edge_relation):
    row = edge_index[0].astype(jnp.int32)
    col = edge_index[1].astype(jnp.int32)
    rel = edge_relation.astype(jnp.int32)

    mtab, gidx = pl.pallas_call(
        _msg_body,
        grid=(NBLK,),
        in_specs=[
            pl.BlockSpec((NB, D), lambda i: (i, 0)),
            pl.BlockSpec((NB, 1), lambda i: (i, 0)),
            pl.BlockSpec((K, D, D), lambda i: (0, 0, 0)),
            pl.BlockSpec((E // D, D), lambda i: (0, 0)),
            pl.BlockSpec((E // D, D), lambda i: (0, 0)),
        ],
        out_specs=[
            pl.BlockSpec((K, NB, D), lambda i: (0, i, 0)),
            pl.BlockSpec((2, E // D, D), lambda i: (0, 0, 0)),
        ],
        out_shape=[
            jax.ShapeDtypeStruct((K, N, D), jnp.float32),
            jax.ShapeDtypeStruct((2, E // D, D), jnp.int32),
        ],
    )(features, norm, W.reshape(K, D, D),
      col.reshape(E // D, D), rel.reshape(E // D, D))

    mesh = plsc.VectorSubcoreMesh(core_axis_name="c", subcore_axis_name="s")
    acc = pl.kernel(
        _sc_body,
        out_type=jax.ShapeDtypeStruct((NHEADS * NPAD, D), jnp.float32),
        mesh=mesh,
        scratch_types=[
            pltpu.VMEM((IB, CH), jnp.int32),
            pltpu.VMEM((IB, CH), jnp.int32),
            pltpu.VMEM((IB, CH), jnp.int32),
            pltpu.VMEM((IB, CH), jnp.int32),
            pltpu.VMEM((CH, D), jnp.float32),
            pltpu.VMEM((CH, D), jnp.float32),
            pltpu.VMEM_SHARED((NPAD, D), jnp.float32),
            pltpu.SemaphoreType.DMA,
            pltpu.SemaphoreType.DMA,
            pltpu.SemaphoreType.DMA,
            pltpu.SemaphoreType.DMA,
        ],
    )(mtab.reshape(K * N, D), gidx.reshape(NHEADS * E // CH, CH),
      row.reshape(E // CH, CH))

    out = pl.pallas_call(
        _final_body,
        grid=(NBLK,),
        in_specs=[
            pl.BlockSpec((NHEADS, NB, D), lambda i: (0, i, 0)),
            pl.BlockSpec((NB, 1), lambda i: (i, 0)),
        ],
        out_specs=pl.BlockSpec((NB, D), lambda i: (i, 0)),
        out_shape=jax.ShapeDtypeStruct((N, D), jnp.float32),
    )(acc.reshape(NHEADS, NPAD, D), norm)
    return out
